# baseline (speedup 1.0000x reference)
import functools

import jax
import jax.numpy as jnp
from jax import lax
from jax.experimental import pallas as pl
from jax.experimental.pallas import tpu as pltpu
from jax.experimental.pallas import tpu_sc as plsc

_LANES = 16
_NUM_WORKERS = 32


def _sc_embed(word_emb, pos_emb, ids, pids, n_rows, hidden, chunk):
    rows_per_w = n_rows // _NUM_WORKERS
    n_chunks = rows_per_w // chunk
    vregs_per_chunk = chunk * hidden // _LANES

    mesh = plsc.VectorSubcoreMesh(core_axis_name="c", subcore_axis_name="s")

    @functools.partial(
        pl.kernel,
        out_type=jax.ShapeDtypeStruct((n_rows, hidden), jnp.float32),
        mesh=mesh,
        scratch_types=[
            pltpu.VMEM((chunk,), jnp.int32),
            pltpu.VMEM((chunk,), jnp.int32),
            pltpu.VMEM((chunk, hidden), jnp.float32),
            pltpu.VMEM((chunk, hidden), jnp.float32),
            pltpu.SemaphoreType.DMA,
        ],
    )
    def body(word_hbm, pos_hbm, ids_hbm, pids_hbm, out_hbm,
             idx_w, idx_p, wbuf, pbuf, sem):
        wid = lax.axis_index("s") * 2 + lax.axis_index("c")
        w_base = wid * rows_per_w

        def chunk_body(g, _):
            base = w_base + g * chunk
            pltpu.sync_copy(ids_hbm.at[pl.ds(base, chunk)], idx_w)
            pltpu.sync_copy(pids_hbm.at[pl.ds(base, chunk)], idx_p)
            cw = pltpu.async_copy(word_hbm.at[idx_w], wbuf, sem)
            cp = pltpu.async_copy(pos_hbm.at[idx_p], pbuf, sem)
            cw.wait()
            cp.wait()

            def add_body(i, _):
                r = i // (hidden // _LANES)
                j = i % (hidden // _LANES)
                sl = pl.ds(j * _LANES, _LANES)
                wbuf[r, sl] = wbuf[r, sl] + pbuf[r, sl]
                return 0

            lax.fori_loop(0, vregs_per_chunk, add_body, 0)
            pltpu.sync_copy(wbuf, out_hbm.at[pl.ds(base, chunk)])
            return 0

        lax.fori_loop(0, n_chunks, chunk_body, 0)

    return body(word_emb, pos_emb, ids, pids)


def kernel(input_ids, position_ids, word_emb, pos_emb):
    batch, seq = input_ids.shape
    hidden = word_emb.shape[1]
    n_rows = batch * seq

    ids = input_ids.T.reshape(n_rows).astype(jnp.int32)
    pids = position_ids.T.reshape(n_rows).astype(jnp.int32)

    out = _sc_embed(word_emb.astype(jnp.float32), pos_emb.astype(jnp.float32),
                    ids, pids, n_rows, hidden, chunk=32)
    return out.reshape(seq, batch, hidden)


# speedup vs baseline: 1.8683x; 1.8683x over previous
import functools

import jax
import jax.numpy as jnp
from jax import lax
from jax.experimental import pallas as pl
from jax.experimental.pallas import tpu as pltpu
from jax.experimental.pallas import tpu_sc as plsc

_LANES = 16
_NUM_WORKERS = 32
_NBUF = 2


def _sc_embed(word_emb, pos_emb, ids, pids, n_rows, hidden, chunk):
    rows_per_w = n_rows // _NUM_WORKERS
    n_chunks = rows_per_w // chunk
    flat = chunk * hidden
    vregs = flat // _LANES

    mesh = plsc.VectorSubcoreMesh(core_axis_name="c", subcore_axis_name="s")

    scratch = []
    for _ in range(_NBUF):
        scratch += [
            pltpu.VMEM((chunk,), jnp.int32),
            pltpu.VMEM((chunk,), jnp.int32),
            pltpu.VMEM((chunk, hidden), jnp.float32),
            pltpu.VMEM((chunk, hidden), jnp.float32),
            pltpu.VMEM((chunk, hidden), jnp.float32),
            pltpu.SemaphoreType.DMA,
            pltpu.SemaphoreType.DMA,
        ]

    @functools.partial(
        pl.kernel,
        out_type=jax.ShapeDtypeStruct((n_rows, hidden), jnp.float32),
        mesh=mesh,
        scratch_types=scratch,
    )
    def body(word_hbm, pos_hbm, ids_hbm, pids_hbm, out_hbm, *bufs):
        sets = [bufs[i * 7:(i + 1) * 7] for i in range(_NBUF)]
        wid = lax.axis_index("s") * 2 + lax.axis_index("c")
        w_base = wid * rows_per_w

        def issue_gather(b, g):
            idx_w, idx_p, wbuf, pbuf, _, gsem, _ = sets[b]
            base = w_base + g * chunk
            pltpu.sync_copy(ids_hbm.at[pl.ds(base, chunk)], idx_w)
            pltpu.sync_copy(pids_hbm.at[pl.ds(base, chunk)], idx_p)
            pltpu.async_copy(word_hbm.at[idx_w], wbuf,
                             gsem)
            pltpu.async_copy(pos_hbm.at[idx_p], pbuf,
                             gsem)

        def wait_gather(b):
            idx_w, idx_p, wbuf, pbuf, _, gsem, _ = sets[b]
            pltpu.make_async_copy(word_hbm.at[idx_w],
                                  wbuf, gsem).wait()
            pltpu.make_async_copy(pos_hbm.at[idx_p],
                                  pbuf, gsem).wait()

        def issue_out(b, g):
            _, _, _, _, obuf, _, osem = sets[b]
            base = w_base + g * chunk
            pltpu.async_copy(obuf,
                             out_hbm.at[pl.ds(base, chunk)], osem)

        def wait_out(b, g):
            _, _, _, _, obuf, _, osem = sets[b]
            base = w_base + g * chunk
            pltpu.make_async_copy(obuf,
                                  out_hbm.at[pl.ds(base, chunk)], osem).wait()

        def compute(b):
            _, _, wbuf, pbuf, obuf, _, _ = sets[b]
            vregs_per_row = hidden // _LANES

            def add_body(i):
                r = i // vregs_per_row
                sl = pl.ds((i % vregs_per_row) * _LANES, _LANES)
                obuf[r, sl] = wbuf[r, sl] + pbuf[r, sl]

            plsc.parallel_loop(0, vregs, 1, unroll=8)(add_body)

        issue_gather(0, 0)
        issue_gather(1, 1)

        def outer(t, _):
            for b in range(_NBUF):
                g = t * _NBUF + b
                wait_gather(b)

                @pl.when(g >= _NBUF)
                def _():
                    wait_out(b, g - _NBUF)

                compute(b)

                @pl.when(g + _NBUF < n_chunks)
                def _():
                    issue_gather(b, g + _NBUF)

                issue_out(b, g)
            return 0

        lax.fori_loop(0, n_chunks // _NBUF, outer, 0)
        wait_out(0, n_chunks - 2)
        wait_out(1, n_chunks - 1)

    return body(word_emb, pos_emb, ids, pids)


def kernel(input_ids, position_ids, word_emb, pos_emb):
    batch, seq = input_ids.shape
    hidden = word_emb.shape[1]
    n_rows = batch * seq

    ids = input_ids.T.reshape(n_rows).astype(jnp.int32)
    pids = position_ids.T.reshape(n_rows).astype(jnp.int32)

    out = _sc_embed(word_emb.astype(jnp.float32), pos_emb.astype(jnp.float32),
                    ids, pids, n_rows, hidden, chunk=16)
    return out.reshape(seq, batch, hidden)


# speedup vs baseline: 4.0138x; 2.1484x over previous
import functools

import jax
import jax.numpy as jnp
from jax import lax
from jax.experimental import pallas as pl
from jax.experimental.pallas import tpu as pltpu
from jax.experimental.pallas import tpu_sc as plsc

_LANES = 16
_NUM_WORKERS = 32
_NBUF = 2


def _sc_embed(word_emb, pos_emb, ids, pids, seq, batch, hidden, chunk):
    s_span = seq * batch // _NUM_WORKERS
    n_blocks = _NUM_WORKERS // batch
    n_chunks = s_span // chunk
    vregs = chunk * hidden // _LANES
    vregs_per_row = hidden // _LANES

    mesh = plsc.VectorSubcoreMesh(core_axis_name="c", subcore_axis_name="s")

    scratch = [
        pltpu.VMEM((s_span,), jnp.int32),
        pltpu.VMEM((s_span,), jnp.int32),
    ]
    for _ in range(_NBUF):
        scratch += [
            pltpu.VMEM((chunk, hidden), jnp.float32),
            pltpu.VMEM((chunk, hidden), jnp.float32),
            pltpu.VMEM((chunk, hidden), jnp.float32),
            pltpu.SemaphoreType.DMA,
            pltpu.SemaphoreType.DMA,
        ]

    @functools.partial(
        pl.kernel,
        out_type=jax.ShapeDtypeStruct((seq, batch, hidden), jnp.float32),
        mesh=mesh,
        scratch_types=scratch,
    )
    def body(word_hbm, pos_hbm, ids_hbm, pids_hbm, out_hbm,
             idw_all, idp_all, *bufs):
        sets = [bufs[i * 5:(i + 1) * 5] for i in range(_NBUF)]
        wid = lax.axis_index("s") * 2 + lax.axis_index("c")
        b = wid % batch
        s0w = (wid // batch) * s_span

        pltpu.sync_copy(ids_hbm.at[b, pl.ds(s0w, s_span)], idw_all)
        pltpu.sync_copy(pids_hbm.at[b, pl.ds(s0w, s_span)], idp_all)

        def issue_gather(bb, g):
            wbuf, pbuf, _, gsem, _ = sets[bb]
            off = g * chunk
            pltpu.async_copy(word_hbm.at[idw_all.at[pl.ds(off, chunk)]],
                             wbuf, gsem)
            pltpu.async_copy(pos_hbm.at[idp_all.at[pl.ds(off, chunk)]],
                             pbuf, gsem)

        def wait_gather(bb, g):
            wbuf, pbuf, _, gsem, _ = sets[bb]
            off = g * chunk
            pltpu.make_async_copy(word_hbm.at[idw_all.at[pl.ds(off, chunk)]],
                                  wbuf, gsem).wait()
            pltpu.make_async_copy(pos_hbm.at[idp_all.at[pl.ds(off, chunk)]],
                                  pbuf, gsem).wait()

        def issue_out(bb, g):
            _, _, obuf, _, osem = sets[bb]
            s_base = s0w + g * chunk
            pltpu.async_copy(obuf, out_hbm.at[pl.ds(s_base, chunk), b], osem)

        def wait_out(bb, g):
            _, _, obuf, _, osem = sets[bb]
            s_base = s0w + g * chunk
            pltpu.make_async_copy(obuf, out_hbm.at[pl.ds(s_base, chunk), b],
                                  osem).wait()

        def compute(bb):
            wbuf, pbuf, obuf, _, _ = sets[bb]

            def add_body(i):
                r = i // vregs_per_row
                sl = pl.ds((i % vregs_per_row) * _LANES, _LANES)
                obuf[r, sl] = wbuf[r, sl] + pbuf[r, sl]

            plsc.parallel_loop(0, vregs, 1, unroll=8)(add_body)

        issue_gather(0, 0)
        issue_gather(1, 1)

        def outer(t, _):
            for bb in range(_NBUF):
                g = t * _NBUF + bb
                wait_gather(bb, g)

                @pl.when(g >= _NBUF)
                def _():
                    wait_out(bb, g - _NBUF)

                compute(bb)

                @pl.when(g + _NBUF < n_chunks)
                def _():
                    issue_gather(bb, g + _NBUF)

                issue_out(bb, g)
            return 0

        lax.fori_loop(0, n_chunks // _NBUF, outer, 0)
        wait_out(0, n_chunks - 2)
        wait_out(1, n_chunks - 1)

    return body(word_emb, pos_emb, ids, pids)


def kernel(input_ids, position_ids, word_emb, pos_emb):
    batch, seq = input_ids.shape
    hidden = word_emb.shape[1]

    out = _sc_embed(word_emb, pos_emb,
                    input_ids.astype(jnp.int32),
                    position_ids.astype(jnp.int32),
                    seq, batch, hidden, chunk=16)
    return out
